# baseline (speedup 1.0000x reference)
import jax
import jax.numpy as jnp
from jax.experimental import pallas as pl
from jax.experimental.pallas import tpu as pltpu

_BM = 256
_NK = 2


def _pathcon_body(x_ref, w_ref, b_ref, scores_ref, sig_ref):
    k = pl.program_id(1)
    nk = pl.num_programs(1)
    bk = x_ref.shape[1]
    w_chunk = w_ref[:, pl.ds(pl.multiple_of(k * bk, bk), bk)]
    acc = jax.lax.dot_general(
        w_chunk, x_ref[...],
        dimension_numbers=(((1,), (1,)), ((), ())),
        preferred_element_type=jnp.float32,
    )

    @pl.when(k == 0)
    def _init():
        scores_ref[...] = acc

    @pl.when(k > 0)
    def _accum():
        scores_ref[...] += acc

    @pl.when(k == nk - 1)
    def _epilogue():
        scores = scores_ref[...] + b_ref[...]
        scores_ref[...] = scores
        sig_ref[...] = jax.nn.sigmoid(scores)


def kernel(path_features, labels, W, b):
    del labels
    batch, n_paths = path_features.shape
    n_rel = W.shape[0]
    b2 = b.reshape(n_rel, 1)
    bk = n_paths // _NK

    grid = (batch // _BM, _NK)
    out_shape = [
        jax.ShapeDtypeStruct((n_rel, batch), jnp.float32),
        jax.ShapeDtypeStruct((n_rel, batch), jnp.float32),
    ]
    scores_t, sig_t = pl.pallas_call(
        _pathcon_body,
        grid=grid,
        in_specs=[
            pl.BlockSpec((_BM, bk), lambda i, k: (i, k)),
            pl.BlockSpec((n_rel, n_paths), lambda i, k: (0, 0)),
            pl.BlockSpec((n_rel, 1), lambda i, k: (0, 0)),
        ],
        out_specs=[
            pl.BlockSpec((n_rel, _BM), lambda i, k: (0, i)),
            pl.BlockSpec((n_rel, _BM), lambda i, k: (0, i)),
        ],
        out_shape=out_shape,
        compiler_params=pltpu.CompilerParams(
            dimension_semantics=("parallel", "arbitrary"),
        ),
    )(path_features, W, b2)
    return (jnp.swapaxes(scores_t, 0, 1), jnp.swapaxes(sig_t, 0, 1))


# speedup vs baseline: 1.2374x; 1.2374x over previous
import jax
import jax.numpy as jnp
from jax.experimental import pallas as pl
from jax.experimental.pallas import tpu as pltpu

_BM = 256


def _pathcon_body(x_ref, w_ref, b_ref, scores_ref, sig_ref):
    acc = jax.lax.dot_general(
        w_ref[...], x_ref[...],
        dimension_numbers=(((1,), (1,)), ((), ())),
        preferred_element_type=jnp.float32,
    )
    scores = acc + b_ref[...]
    scores_ref[...] = scores
    sig_ref[...] = jax.nn.sigmoid(scores)


def kernel(path_features, labels, W, b):
    del labels
    batch, n_paths = path_features.shape
    n_rel = W.shape[0]
    b2 = b.reshape(n_rel, 1)

    grid = (batch // _BM,)
    out_shape = [
        jax.ShapeDtypeStruct((n_rel, batch), jnp.float32),
        jax.ShapeDtypeStruct((n_rel, batch), jnp.float32),
    ]
    scores_t, sig_t = pl.pallas_call(
        _pathcon_body,
        grid=grid,
        in_specs=[
            pl.BlockSpec((_BM, n_paths), lambda i: (i, 0)),
            pl.BlockSpec((n_rel, n_paths), lambda i: (0, 0)),
            pl.BlockSpec((n_rel, 1), lambda i: (0, 0)),
        ],
        out_specs=[
            pl.BlockSpec((n_rel, _BM), lambda i: (0, i)),
            pl.BlockSpec((n_rel, _BM), lambda i: (0, i)),
        ],
        out_shape=out_shape,
        compiler_params=pltpu.CompilerParams(
            dimension_semantics=("arbitrary",),
        ),
    )(path_features, W, b2)
    return (jnp.swapaxes(scores_t, 0, 1), jnp.swapaxes(sig_t, 0, 1))
